# baseline (speedup 1.0000x reference)
import functools

import jax
import jax.numpy as jnp
from jax import lax
from jax.experimental import pallas as pl
from jax.experimental.pallas import tpu as pltpu
from jax.experimental.pallas import tpu_sc as plsc

N = 10000
E = 320000
D = 128
NCLS = 16

NC = 2
NS = 16
K = 128
CHE = 160
EPT = K * CHE
EPAD = NS * EPT

BLK = 512
NPAD = 10240
HALF = NPAD // NC
DUMMY = 128
ACC_ROWS = HALF + DUMMY
ART = ACC_ROWS // NS
ORT = HALF // NS
GRID = NPAD // BLK


NBUF = 4


@functools.cache
def _sc_spmm_kernel():
    mesh = plsc.VectorSubcoreMesh(
        core_axis_name="c", subcore_axis_name="s",
        num_cores=NC, num_subcores=NS)
    return functools.partial(
        pl.kernel,
        out_type=jax.ShapeDtypeStruct((NC, HALF, D), jnp.float32),
        mesh=mesh,
        scratch_types=[
            pltpu.VMEM((CHE // 2, K), jnp.int32),
            pltpu.VMEM((CHE // 2, K), jnp.int32),
            [pltpu.VMEM((K, D), jnp.float32) for _ in range(NBUF)],
            pltpu.VMEM_SHARED((ACC_ROWS, D), jnp.float32),
            [pltpu.SemaphoreType.DMA for _ in range(NBUF)],
            [pltpu.SemaphoreType.DMA for _ in range(NBUF)],
        ],
    )(_sc_spmm_body)


def _sc_spmm_call(hp, srcs, dstm, zer128):
    return _sc_spmm_kernel()(hp, srcs, dstm, zer128)


def _sc_spmm_body(hp_hbm, srcs_hbm, dstm_hbm, zer_hbm, out_hbm,
                  src_v, dst_v, g, acc, gs, ss):
    c = lax.axis_index("c")
    s = lax.axis_index("s")
    def zslab(t, _):
        pltpu.sync_copy(zer_hbm, acc.at[pl.ds(s * ART + t * 82, 82)])
        return 0
    lax.fori_loop(0, ART // 82, zslab, 0)
    plsc.subcore_barrier()
    CH2 = CHE // 2
    for half in range(2):
        pltpu.sync_copy(srcs_hbm.at[s, pl.ds(half * CH2, CH2)], src_v)
        pltpu.sync_copy(dstm_hbm.at[c, s, pl.ds(half * CH2, CH2)], dst_v)
        for u in range(NBUF):
            pltpu.async_copy(hp_hbm.at[src_v.at[u]], g[u], gs[u])
        def step(j, _):
            for u in range(NBUF):
                k = NBUF * j + u
                pltpu.make_async_copy(hp_hbm.at[src_v.at[k]], g[u], gs[u]).wait()
                pltpu.async_copy(g[u], acc.at[dst_v.at[k]], ss[u], add=True)
            for u in range(NBUF):
                k = NBUF * j + u
                pltpu.make_async_copy(g[u], acc.at[dst_v.at[k]], ss[u]).wait()
                @pl.when(k + NBUF < CH2)
                def _():
                    pltpu.async_copy(hp_hbm.at[src_v.at[k + NBUF]], g[u], gs[u])
            return 0
        lax.fori_loop(0, CH2 // NBUF, step, 0)
    plsc.subcore_barrier()
    pltpu.sync_copy(
        acc.at[pl.ds(s * ORT, ORT)],
        out_hbm.at[c, pl.ds(s * ORT, ORT)],
    )


RMBLK = 256


def _tc_remap_body(dst_ref, o_ref):
    v = dst_ref[...]
    for core in range(NC):
        base = core * HALF
        ok = (v >= base) & (v < base + HALF)
        o_ref[core] = jnp.where(ok, v - base, HALF + (v & (DUMMY - 1)))


_tc_remap = pl.pallas_call(
    _tc_remap_body,
    grid=(NS * CHE // RMBLK,),
    in_specs=[pl.BlockSpec((RMBLK, K), lambda i: (i, 0))],
    out_specs=pl.BlockSpec((NC, RMBLK, K), lambda i: (0, i, 0)),
    out_shape=jax.ShapeDtypeStruct((NC, NS * CHE, K), jnp.int32),
)


def _dinv_block(deg_ref, i):
    deg = deg_ref[:, 0:1] + 1.0
    dinv = lax.rsqrt(deg)
    rows = lax.broadcasted_iota(jnp.int32, (BLK, 1), 0) + i * BLK
    return jnp.where(rows < N, dinv, 0.0)


def _tc_pre_body(deg_ref, x_ref, w_ref, o_ref):
    i = pl.program_id(0)
    dinv = _dinv_block(deg_ref, i)
    h = jnp.dot(x_ref[...], w_ref[...], preferred_element_type=jnp.float32)
    o_ref[...] = h * dinv


def _tc_mid_body(deg_ref, p_ref, hp_ref, b_ref, w_ref, z_ref, o_ref):
    i = pl.program_id(0)
    dinv = _dinv_block(deg_ref, i)
    tot = p_ref[...] + hp_ref[...]
    z = jnp.maximum(tot * dinv + b_ref[...], 0.0)
    z_ref[...] = z
    o_ref[...] = jnp.dot(z, w_ref[...], preferred_element_type=jnp.float32) * dinv


def _tc_final_body(z_ref, wl_ref, bl_ref, o_ref):
    logits = jnp.dot(z_ref[...], wl_ref[...], preferred_element_type=jnp.float32) + bl_ref[...]
    m = jnp.max(logits, axis=-1, keepdims=True)
    lse = jnp.log(jnp.sum(jnp.exp(logits - m), axis=-1, keepdims=True)) + m
    o_ref[...] = logits - lse


_deg_spec = pl.BlockSpec((BLK, D), lambda i: (i, 0))
_row_spec = pl.BlockSpec((BLK, D), lambda i: (i, 0))
_w_spec = pl.BlockSpec((D, D), lambda i: (0, 0))
_b_spec = pl.BlockSpec((1, D), lambda i: (0, 0))

_hp_shape = jax.ShapeDtypeStruct((NPAD, D), jnp.float32)

_tc_pre = pl.pallas_call(
    _tc_pre_body,
    grid=(GRID,),
    in_specs=[_deg_spec, _row_spec, _w_spec],
    out_specs=_row_spec,
    out_shape=_hp_shape,
)

_tc_mid = pl.pallas_call(
    _tc_mid_body,
    grid=(GRID,),
    in_specs=[_deg_spec, _row_spec, _row_spec, _b_spec, _w_spec],
    out_specs=[_row_spec, _row_spec],
    out_shape=[_hp_shape, _hp_shape],
)

_tc_final = pl.pallas_call(
    _tc_final_body,
    grid=(GRID,),
    in_specs=[
        _row_spec,
        pl.BlockSpec((D, NCLS), lambda i: (0, 0)),
        pl.BlockSpec((1, NCLS), lambda i: (0, 0)),
    ],
    out_specs=pl.BlockSpec((BLK, NCLS), lambda i: (i, 0)),
    out_shape=jax.ShapeDtypeStruct((NPAD, NCLS), jnp.float32),
)


def kernel(x, edge_index, W1, b1, W2, b2, W3, b3, Wl, bl):
    src = edge_index[0]
    dst = edge_index[1]
    pad = jnp.full((EPAD - E,), N, jnp.int32)
    srcs = jnp.concatenate([src, pad]).reshape(NS, CHE, K)
    dsts = jnp.concatenate([dst, pad]).reshape(NS, CHE, K)
    x_pad = jnp.concatenate([x, jnp.zeros((NPAD - N, D), x.dtype)], axis=0)
    zer128 = jnp.zeros((82, D), jnp.float32)
    ones_full = jnp.ones((NPAD, D), jnp.float32)
    srcs0 = jnp.zeros((NS, CHE, K), jnp.int32)

    dstm = _tc_remap(dsts.reshape(NS * CHE, K)).reshape(NC, NS, CHE, K)
    deg = _sc_spmm_call(ones_full, srcs0, dstm, zer128).reshape(NPAD, D)
    h1p = _tc_pre(deg, x_pad, W1)

    bs = jnp.stack([b1.reshape(1, D), b2.reshape(1, D), b3.reshape(1, D)])
    Wn = jnp.stack([W2, W3, jnp.zeros_like(W3)])
    trip = 3 + (x[0, 0] != x[0, 0]).astype(jnp.int32)

    def body(i, carry):
        hp, _ = carry
        p = _sc_spmm_call(hp, srcs, dstm, zer128).reshape(NPAD, D)
        b_l = lax.dynamic_index_in_dim(bs, i, 0, keepdims=False)
        Wn_l = lax.dynamic_index_in_dim(Wn, i, 0, keepdims=False)
        z, hp_next = _tc_mid(deg, p, hp, b_l, Wn_l)
        return (hp_next, z)

    _, z3 = lax.fori_loop(0, trip, body, (h1p, h1p))
    out_pad = _tc_final(z3, Wl, bl.reshape(1, NCLS))
    return out_pad[:N]


# speedup vs baseline: 8.5665x; 8.5665x over previous
import functools

import jax
import jax.numpy as jnp
from jax import lax
from jax.experimental import pallas as pl
from jax.experimental.pallas import tpu as pltpu
from jax.experimental.pallas import tpu_sc as plsc

N = 10000
E = 320000
D = 128
NCLS = 16

NC = 2
NS = 16
K = 128
CHE = 160
EPT = K * CHE
EPAD = NS * EPT

BLK = 512
NPAD = 10240
HALF = NPAD // NC
DUMMY = 128
ACC_ROWS = HALF + DUMMY
ART = ACC_ROWS // NS
ORT = HALF // NS
GRID = NPAD // BLK


@functools.cache
def _sc_spmm_kernel():
    mesh = plsc.VectorSubcoreMesh(
        core_axis_name="c", subcore_axis_name="s",
        num_cores=NC, num_subcores=NS)
    return functools.partial(
        pl.kernel,
        out_type=jax.ShapeDtypeStruct((NC, HALF, D), jnp.float32),
        mesh=mesh,
        scratch_types=[
            pltpu.VMEM((CHE, K), jnp.int32),
            pltpu.VMEM((CHE, K), jnp.int32),
            pltpu.VMEM((K, D), jnp.float32),
            pltpu.VMEM((K, D), jnp.float32),
            pltpu.VMEM_SHARED((ACC_ROWS, D), jnp.float32),
            pltpu.SemaphoreType.DMA,
            pltpu.SemaphoreType.DMA,
        ],
    )(_sc_spmm_body)


def _sc_spmm_call(hp, srcs, dstm, zer128):
    return _sc_spmm_kernel()(hp, srcs, dstm, zer128)


def _sc_spmm_body(hp_hbm, srcs_hbm, dstm_hbm, zer_hbm, out_hbm,
                  src_v, dst_v, g0, g1, acc, sem0, sem1):
    c = lax.axis_index("c")
    s = lax.axis_index("s")
    def zslab(t, _):
        pltpu.sync_copy(zer_hbm, acc.at[pl.ds(s * ART + t * 82, 82)])
        return 0
    lax.fori_loop(0, ART // 82, zslab, 0)
    pltpu.sync_copy(srcs_hbm.at[s], src_v)
    pltpu.sync_copy(dstm_hbm.at[c, s], dst_v)
    plsc.subcore_barrier()
    pltpu.async_copy(hp_hbm.at[src_v.at[0]], g0, sem0)
    def step(j, _):
        pltpu.async_copy(hp_hbm.at[src_v.at[2 * j + 1]], g1, sem1)
        pltpu.make_async_copy(hp_hbm.at[src_v.at[2 * j]], g0, sem0).wait()
        pltpu.sync_copy(g0, acc.at[dst_v.at[2 * j]], add=True)
        @pl.when(j < CHE // 2 - 1)
        def _():
            pltpu.async_copy(hp_hbm.at[src_v.at[2 * j + 2]], g0, sem0)
        pltpu.make_async_copy(hp_hbm.at[src_v.at[2 * j + 1]], g1, sem1).wait()
        pltpu.sync_copy(g1, acc.at[dst_v.at[2 * j + 1]], add=True)
        return 0
    lax.fori_loop(0, CHE // 2, step, 0)
    plsc.subcore_barrier()
    pltpu.sync_copy(
        acc.at[pl.ds(s * ORT, ORT)],
        out_hbm.at[c, pl.ds(s * ORT, ORT)],
    )


RMBLK = 256


def _tc_remap_body(dst_ref, o_ref):
    v = dst_ref[...]
    for core in range(NC):
        base = core * HALF
        ok = (v >= base) & (v < base + HALF)
        o_ref[core] = jnp.where(ok, v - base, HALF + (v & (DUMMY - 1)))


_tc_remap = pl.pallas_call(
    _tc_remap_body,
    grid=(NS * CHE // RMBLK,),
    in_specs=[pl.BlockSpec((RMBLK, K), lambda i: (i, 0))],
    out_specs=pl.BlockSpec((NC, RMBLK, K), lambda i: (0, i, 0)),
    out_shape=jax.ShapeDtypeStruct((NC, NS * CHE, K), jnp.int32),
)


def _dinv_block(deg_ref, i):
    deg = deg_ref[:, 0:1] + 1.0
    dinv = lax.rsqrt(deg)
    rows = lax.broadcasted_iota(jnp.int32, (BLK, 1), 0) + i * BLK
    return jnp.where(rows < N, dinv, 0.0)


def _tc_pre_body(deg_ref, x_ref, w_ref, o_ref):
    i = pl.program_id(0)
    dinv = _dinv_block(deg_ref, i)
    h = jnp.dot(x_ref[...], w_ref[...], preferred_element_type=jnp.float32)
    o_ref[...] = h * dinv


def _tc_mid_body(deg_ref, p_ref, hp_ref, b_ref, w_ref, z_ref, o_ref):
    i = pl.program_id(0)
    dinv = _dinv_block(deg_ref, i)
    tot = p_ref[...] + hp_ref[...]
    z = jnp.maximum(tot * dinv + b_ref[...], 0.0)
    z_ref[...] = z
    o_ref[...] = jnp.dot(z, w_ref[...], preferred_element_type=jnp.float32) * dinv


def _tc_final_body(z_ref, wl_ref, bl_ref, o_ref):
    logits = jnp.dot(z_ref[...], wl_ref[...], preferred_element_type=jnp.float32) + bl_ref[...]
    m = jnp.max(logits, axis=-1, keepdims=True)
    lse = jnp.log(jnp.sum(jnp.exp(logits - m), axis=-1, keepdims=True)) + m
    o_ref[...] = logits - lse


_deg_spec = pl.BlockSpec((BLK, 16), lambda i: (i, 0))
_row_spec = pl.BlockSpec((BLK, D), lambda i: (i, 0))
_w_spec = pl.BlockSpec((D, D), lambda i: (0, 0))
_b_spec = pl.BlockSpec((1, D), lambda i: (0, 0))

_hp_shape = jax.ShapeDtypeStruct((NPAD, D), jnp.float32)

_tc_pre = pl.pallas_call(
    _tc_pre_body,
    grid=(GRID,),
    in_specs=[_deg_spec, _row_spec, _w_spec],
    out_specs=_row_spec,
    out_shape=_hp_shape,
)

_tc_mid = pl.pallas_call(
    _tc_mid_body,
    grid=(GRID,),
    in_specs=[_deg_spec, _row_spec, _row_spec, _b_spec, _w_spec],
    out_specs=[_row_spec, _row_spec],
    out_shape=[_hp_shape, _hp_shape],
)

_tc_final = pl.pallas_call(
    _tc_final_body,
    grid=(GRID,),
    in_specs=[
        _row_spec,
        pl.BlockSpec((D, NCLS), lambda i: (0, 0)),
        pl.BlockSpec((1, NCLS), lambda i: (0, 0)),
    ],
    out_specs=pl.BlockSpec((BLK, NCLS), lambda i: (i, 0)),
    out_shape=jax.ShapeDtypeStruct((NPAD, NCLS), jnp.float32),
)


def kernel(x, edge_index, W1, b1, W2, b2, W3, b3, Wl, bl):
    src = edge_index[0]
    dst = edge_index[1]
    pad = jnp.full((EPAD - E,), N, jnp.int32)
    srcs = jnp.concatenate([src, pad]).reshape(NS, CHE, K)
    dsts = jnp.concatenate([dst, pad]).reshape(NS, CHE, K)
    x_pad = jnp.concatenate([x, jnp.zeros((NPAD - N, D), x.dtype)], axis=0)
    zer128 = jnp.zeros((82, D), jnp.float32)
    ones_full = jnp.ones((NPAD, D), jnp.float32)

    dstm = _tc_remap(dsts.reshape(NS * CHE, K)).reshape(NC, NS, CHE, K)
    deg = _sc_spmm_call(ones_full, srcs, dstm, zer128).reshape(NPAD, D)[:, :16]
    h1p = _tc_pre(deg, x_pad, W1)

    bs = jnp.stack([b1.reshape(1, D), b2.reshape(1, D), b3.reshape(1, D)])
    Wn = jnp.stack([W2, W3, jnp.zeros_like(W3)])
    trip = 3 + (x[0, 0] != x[0, 0]).astype(jnp.int32)

    def body(i, carry):
        hp, _ = carry
        p = _sc_spmm_call(hp, srcs, dstm, zer128).reshape(NPAD, D)
        b_l = lax.dynamic_index_in_dim(bs, i, 0, keepdims=False)
        Wn_l = lax.dynamic_index_in_dim(Wn, i, 0, keepdims=False)
        z, hp_next = _tc_mid(deg, p, hp, b_l, Wn_l)
        return (hp_next, z)

    _, z3 = lax.fori_loop(0, trip, body, (h1p, h1p))
    out_pad = _tc_final(z3, Wl, bl.reshape(1, NCLS))
    return out_pad[:N]
